# baseline (speedup 1.0000x reference)
import functools

import jax
import jax.numpy as jnp
from jax import lax
from jax.experimental import pallas as pl
from jax.experimental.pallas import tpu as pltpu
from jax.experimental.pallas import tpu_sc as plsc

N = 10000
E = 320000
NF = 128
H = 8
OUT = 16
ALPHA = 0.2

NC = 2
NS = 16
NW = NC * NS
EPW = E // NW
B = 80
NCHUNK = EPW // B
RPT = N // NS


def _proj_body(x_ref, wcat_ref, bcat_ref, a1_ref, b1_ref, a2_ref, b2_ref,
               wh_ref, s1_ref, s2_ref):
    xv = x_ref[...]
    wh = jnp.dot(xv, wcat_ref[...], preferred_element_type=jnp.float32,
                 precision=lax.Precision.HIGHEST) + bcat_ref[...]
    wh_ref[...] = wh
    s1_ref[...] = jnp.dot(wh, a1_ref[...], preferred_element_type=jnp.float32,
                          precision=lax.Precision.HIGHEST) + b1_ref[...]
    s2_ref[...] = jnp.dot(wh, a2_ref[...], preferred_element_type=jnp.float32,
                          precision=lax.Precision.HIGHEST) + b2_ref[...]


def _tc_project(x, wcat, bcat, a1, b1, a2, b2):
    mb = 1000
    return pl.pallas_call(
        _proj_body,
        grid=(N // mb,),
        in_specs=[
            pl.BlockSpec((mb, NF), lambda i: (i, 0)),
            pl.BlockSpec((NF, NF), lambda i: (0, 0)),
            pl.BlockSpec((1, NF), lambda i: (0, 0)),
            pl.BlockSpec((NF, 16), lambda i: (0, 0)),
            pl.BlockSpec((1, 16), lambda i: (0, 0)),
            pl.BlockSpec((NF, 16), lambda i: (0, 0)),
            pl.BlockSpec((1, 16), lambda i: (0, 0)),
        ],
        out_specs=[
            pl.BlockSpec((mb, NF), lambda i: (i, 0)),
            pl.BlockSpec((mb, 16), lambda i: (i, 0)),
            pl.BlockSpec((mb, 16), lambda i: (i, 0)),
        ],
        out_shape=[
            jax.ShapeDtypeStruct((N, NF), jnp.float32),
            jax.ShapeDtypeStruct((N, 16), jnp.float32),
            jax.ShapeDtypeStruct((N, 16), jnp.float32),
        ],
    )(x, wcat, bcat, a1, b1, a2, b2)


def _dyn_gather(vec, idx):
    return lax.gather(
        vec, idx[:, None],
        lax.GatherDimensionNumbers(offset_dims=(), collapsed_slice_dims=(0,),
                                   start_index_map=(0,)),
        (1,), mode=lax.GatherScatterMode.PROMISE_IN_BOUNDS)


def _sc_edge_pass(ei, s1_tab, s2_tab, wh, z128, z16):
    mesh = plsc.VectorSubcoreMesh(core_axis_name="c", subcore_axis_name="s")

    @functools.partial(
        pl.kernel,
        out_type=[
            jax.ShapeDtypeStruct((NC, N, NF), jnp.float32),
            jax.ShapeDtypeStruct((NC, N, 16), jnp.float32),
        ],
        mesh=mesh,
        compiler_params=pltpu.CompilerParams(use_tc_tiling_on_sc=False,
                                             needs_layout_passes=False),
        scratch_types=[
            pltpu.VMEM((8, B), jnp.int32),
            pltpu.VMEM((2, B, 16), jnp.float32),
            pltpu.VMEM((2, B, 16), jnp.float32),
            pltpu.VMEM((2, B, NF), jnp.float32),
            pltpu.VMEM((2, B, 16), jnp.float32),
            pltpu.VMEM_SHARED((N, NF), jnp.float32),
            pltpu.VMEM_SHARED((N, 16), jnp.float32),
            pltpu.SemaphoreType.DMA,
            pltpu.SemaphoreType.DMA,
            pltpu.SemaphoreType.DMA,
            pltpu.SemaphoreType.DMA,
            pltpu.SemaphoreType.DMA,
            pltpu.SemaphoreType.DMA,
        ],
    )
    def k(ei_h, s1_h, s2_h, wh_h, z128_h, z16_h, nump_h, denp_h,
          idxb, ssrcb, sdstb, whb, wb, nsh, dsh,
          gsem0, gsem1, ssem0, ssem1, stsem0, stsem1):
        c = lax.axis_index("c")
        s = lax.axis_index("s")
        wid = c * NS + s
        rbase = s * RPT
        gsems = (gsem0, gsem1)
        ssems = (ssem0, ssem1)
        stsems = (stsem0, stsem1)
        pltpu.sync_copy(z128_h.at[pl.ds(rbase, RPT)], nsh.at[pl.ds(rbase, RPT)])
        pltpu.sync_copy(z16_h.at[pl.ds(rbase, RPT)], dsh.at[pl.ds(rbase, RPT)])

        def stage_start(j, par):
            r4 = jnp.bitwise_and(j, 3)
            pltpu.async_copy(ei_h.at[wid, j], idxb.at[pl.ds(2 * r4, 2)],
                             stsems[par])

        def stage_wait(j, par):
            r4 = jnp.bitwise_and(j, 3)
            pltpu.make_async_copy(ei_h.at[wid, j], idxb.at[pl.ds(2 * r4, 2)],
                                  stsems[par]).wait()

        stage_start(0, 0)
        stage_start(1, 1)
        stage_start(2, 0)
        stage_start(3, 1)

        plsc.subcore_barrier()
        lane = lax.iota(jnp.int32, 16)
        lo_idx = jnp.bitwise_and(lane, 7)

        def gather_start(j, slot):
            r4 = jnp.bitwise_and(j, 3)
            si = idxb.at[2 * r4]
            di = idxb.at[2 * r4 + 1]
            pltpu.async_copy(s1_h.at[si], ssrcb.at[slot], gsems[slot])
            pltpu.async_copy(s2_h.at[di], sdstb.at[slot], gsems[slot])
            pltpu.async_copy(wh_h.at[si], whb.at[slot], gsems[slot])

        def gather_wait(j, slot):
            r4 = jnp.bitwise_and(j, 3)
            si = idxb.at[2 * r4]
            di = idxb.at[2 * r4 + 1]
            pltpu.make_async_copy(s1_h.at[si], ssrcb.at[slot],
                                  gsems[slot]).wait()
            pltpu.make_async_copy(s2_h.at[di], sdstb.at[slot],
                                  gsems[slot]).wait()
            pltpu.make_async_copy(wh_h.at[si], whb.at[slot], gsems[slot]).wait()

        def scatter_start(j, slot):
            r4 = jnp.bitwise_and(j, 3)
            di = idxb.at[2 * r4 + 1]
            h1 = pltpu.async_copy(whb.at[slot], nsh.at[di], ssems[slot],
                                  add=True)
            h2 = pltpu.async_copy(wb.at[slot], dsh.at[di], ssems[slot],
                                  add=True)
            return (h1, h2)

        def compute(slot):
            sb = ssrcb.at[slot]
            db = sdstb.at[slot]
            whs = whb.at[slot]
            wbs = wb.at[slot]

            @pl.loop(0, B, step=4)
            def _edge(e0):
                for u in range(4):
                    e = e0 + u
                    t = sb[e, :] + db[e, :]
                    t = jnp.where(t > 0, t, ALPHA * t)
                    w = jnp.exp(t)
                    wbs[e, :] = w
                    wv = _dyn_gather(w, lo_idx)
                    for h in range(H):
                        v = whs[e, pl.ds(h * OUT, OUT)]
                        whs[e, pl.ds(h * OUT, OUT)] = v * wv

        NPAIR = NCHUNK // 2
        stage_wait(0, 0)
        gather_start(0, 0)
        stage_wait(1, 1)
        gather_start(1, 1)

        @pl.loop(0, NPAIR)
        def _pair(p):
            j0 = 2 * p
            j1 = j0 + 1
            gather_wait(j0, 0)
            compute(0)
            sc0 = scatter_start(j0, 0)
            stage_wait(j0 + 2, 0)
            gather_start(j0 + 2, 0)
            gather_wait(j1, 1)
            compute(1)
            sc1 = scatter_start(j1, 1)

            @pl.when(p < NPAIR - 1)
            def _():
                stage_wait(j1 + 2, 1)
                gather_start(j1 + 2, 1)

            sc0[0].wait()
            sc0[1].wait()
            sc1[0].wait()
            sc1[1].wait()

            @pl.when(p < NPAIR - 1)
            def _():
                stage_start(j0 + 4, 0)

            @pl.when(p < NPAIR - 2)
            def _():
                stage_start(j1 + 4, 1)

        gather_wait(NCHUNK - 1, 0)
        compute(0)
        sce = scatter_start(NCHUNK - 1, 0)
        sce[0].wait()
        sce[1].wait()

        plsc.subcore_barrier()
        pltpu.sync_copy(nsh.at[pl.ds(rbase, RPT)],
                        nump_h.at[c].at[pl.ds(rbase, RPT)])
        pltpu.sync_copy(dsh.at[pl.ds(rbase, RPT)],
                        denp_h.at[c].at[pl.ds(rbase, RPT)])

    return k(ei, s1_tab, s2_tab, wh, z128, z16)


def _combine_body(np_ref, dp_ref, e_ref, p_ref, o_ref):
    num = np_ref[0] + np_ref[1]
    den = dp_ref[0] + dp_ref[1]
    den = jnp.where(den == 0.0, 1.0, den)
    rec = 1.0 / den
    res = num * jnp.dot(rec, e_ref[...],
                        preferred_element_type=jnp.float32,
                        precision=lax.Precision.HIGHEST)
    o_ref[...] = jnp.dot(res, p_ref[...],
                         preferred_element_type=jnp.float32,
                         precision=lax.Precision.HIGHEST)


def _tc_combine(nump, denp, eexp, pmat):
    mb = 1000
    return pl.pallas_call(
        _combine_body,
        grid=(N // mb,),
        in_specs=[
            pl.BlockSpec((NC, mb, NF), lambda i: (0, i, 0)),
            pl.BlockSpec((NC, mb, 16), lambda i: (0, i, 0)),
            pl.BlockSpec((16, NF), lambda i: (0, 0)),
            pl.BlockSpec((NF, NF), lambda i: (0, 0)),
        ],
        out_specs=pl.BlockSpec((mb, NF), lambda i: (i, 0)),
        out_shape=jax.ShapeDtypeStruct((N, NF), jnp.float32),
    )(nump, denp, eexp, pmat)


def kernel(x, edge_index, W, W_b, a_w, a_b):
    wcat = jnp.transpose(W, (1, 0, 2)).reshape(NF, NF)
    bcat = W_b.reshape(1, NF)
    kcol = jnp.arange(NF, dtype=jnp.int32)
    perm = (kcol % H) * OUT + kcol // H
    wcat2 = wcat[:, perm]
    bcat2 = bcat[:, perm]
    asrc, adst = a_w[:, :OUT], a_w[:, OUT:]
    eye8 = jnp.eye(H, dtype=jnp.float32)
    a1h = (asrc.T[:, :, None] * eye8[None, :, :]).reshape(NF, H)
    a2h = (adst.T[:, :, None] * eye8[None, :, :]).reshape(NF, H)
    zc = jnp.zeros((NF, H), jnp.float32)
    a1 = jnp.concatenate([a1h, zc], axis=1)
    a2 = jnp.concatenate([a2h, zc], axis=1)
    b1 = jnp.concatenate([a_b, jnp.zeros((H,), jnp.float32)]).reshape(1, 16)
    b2 = jnp.concatenate([jnp.zeros((H,), jnp.float32),
                          jnp.full((H,), -1e30, jnp.float32)]).reshape(1, 16)
    eexp = jnp.concatenate(
        [jnp.tile(eye8, (1, OUT)),
         jnp.zeros((8, NF), jnp.float32)], axis=0)
    pmat = jax.nn.one_hot(perm, NF, dtype=jnp.float32)
    src = edge_index[0].astype(jnp.int32).reshape(NW, NCHUNK, B)
    dst = edge_index[1].astype(jnp.int32).reshape(NW, NCHUNK, B)
    ei = jnp.stack([src, dst], axis=2)
    z128 = jnp.zeros((N, NF), jnp.float32)
    z16 = jnp.zeros((N, 16), jnp.float32)

    wh2, s1_tab, s2_tab = _tc_project(x, wcat2, bcat2, a1, b1, a2, b2)
    nump, denp = _sc_edge_pass(ei, s1_tab, s2_tab, wh2, z128, z16)
    return _tc_combine(nump, denp, eexp, pmat)


# speedup vs baseline: 1.0291x; 1.0291x over previous
import functools

import jax
import jax.numpy as jnp
from jax import lax
from jax.experimental import pallas as pl
from jax.experimental.pallas import tpu as pltpu
from jax.experimental.pallas import tpu_sc as plsc

N = 10000
E = 320000
NF = 128
H = 8
OUT = 16
ALPHA = 0.2

NC = 2
NS = 16
NW = NC * NS
EPW = E // NW
B = 80
NCHUNK = EPW // B
RPT = N // NS


def _proj_body(x_ref, wcat_ref, bcat_ref, a1_ref, b1_ref, a2_ref, b2_ref,
               whf_ref, s2_ref):
    xv = x_ref[...]
    wh = jnp.dot(xv, wcat_ref[...], preferred_element_type=jnp.float32,
                 precision=lax.Precision.HIGHEST) + bcat_ref[...]
    s1 = jnp.dot(wh, a1_ref[...], preferred_element_type=jnp.float32,
                 precision=lax.Precision.HIGHEST) + b1_ref[...]
    whf_ref[...] = jnp.concatenate([wh, s1], axis=1)
    s2_ref[...] = jnp.dot(wh, a2_ref[...], preferred_element_type=jnp.float32,
                          precision=lax.Precision.HIGHEST) + b2_ref[...]


def _tc_project(x, wcat, bcat, a1, b1, a2, b2):
    mb = 1000
    return pl.pallas_call(
        _proj_body,
        grid=(N // mb,),
        in_specs=[
            pl.BlockSpec((mb, NF), lambda i: (i, 0)),
            pl.BlockSpec((NF, NF), lambda i: (0, 0)),
            pl.BlockSpec((1, NF), lambda i: (0, 0)),
            pl.BlockSpec((NF, 16), lambda i: (0, 0)),
            pl.BlockSpec((1, 16), lambda i: (0, 0)),
            pl.BlockSpec((NF, 16), lambda i: (0, 0)),
            pl.BlockSpec((1, 16), lambda i: (0, 0)),
        ],
        out_specs=[
            pl.BlockSpec((mb, NF + 16), lambda i: (i, 0)),
            pl.BlockSpec((mb, 16), lambda i: (i, 0)),
        ],
        out_shape=[
            jax.ShapeDtypeStruct((N, NF + 16), jnp.float32),
            jax.ShapeDtypeStruct((N, 16), jnp.float32),
        ],
    )(x, wcat, bcat, a1, b1, a2, b2)


def _dyn_gather(vec, idx):
    return lax.gather(
        vec, idx[:, None],
        lax.GatherDimensionNumbers(offset_dims=(), collapsed_slice_dims=(0,),
                                   start_index_map=(0,)),
        (1,), mode=lax.GatherScatterMode.PROMISE_IN_BOUNDS)


def _sc_edge_pass(ei, s2_tab, whf, z144):
    mesh = plsc.VectorSubcoreMesh(core_axis_name="c", subcore_axis_name="s")

    @functools.partial(
        pl.kernel,
        out_type=jax.ShapeDtypeStruct((NC, N, NF + 16), jnp.float32),
        mesh=mesh,
        compiler_params=pltpu.CompilerParams(use_tc_tiling_on_sc=False,
                                             needs_layout_passes=False),
        scratch_types=[
            pltpu.VMEM((8, B), jnp.int32),
            pltpu.VMEM((2, B, 16), jnp.float32),
            pltpu.VMEM((2, B, NF + 16), jnp.float32),
            pltpu.VMEM_SHARED((N, NF + 16), jnp.float32),
            pltpu.SemaphoreType.DMA,
            pltpu.SemaphoreType.DMA,
            pltpu.SemaphoreType.DMA,
            pltpu.SemaphoreType.DMA,
            pltpu.SemaphoreType.DMA,
            pltpu.SemaphoreType.DMA,
        ],
    )
    def k(ei_h, s2_h, whf_h, z144_h, nump_h,
          idxb, sdstb, whb, nsh,
          gsem0, gsem1, ssem0, ssem1, stsem0, stsem1):
        c = lax.axis_index("c")
        s = lax.axis_index("s")
        wid = c * NS + s
        rbase = s * RPT
        gsems = (gsem0, gsem1)
        ssems = (ssem0, ssem1)
        stsems = (stsem0, stsem1)
        pltpu.sync_copy(z144_h.at[pl.ds(rbase, RPT)], nsh.at[pl.ds(rbase, RPT)])

        def stage_start(j, par):
            r4 = jnp.bitwise_and(j, 3)
            pltpu.async_copy(ei_h.at[wid, j], idxb.at[pl.ds(2 * r4, 2)],
                             stsems[par])

        def stage_wait(j, par):
            r4 = jnp.bitwise_and(j, 3)
            pltpu.make_async_copy(ei_h.at[wid, j], idxb.at[pl.ds(2 * r4, 2)],
                                  stsems[par]).wait()

        stage_start(0, 0)
        stage_start(1, 1)
        stage_start(2, 0)
        stage_start(3, 1)

        plsc.subcore_barrier()
        lane = lax.iota(jnp.int32, 16)
        lo_idx = jnp.bitwise_and(lane, 7)

        def gather_start(j, slot):
            r4 = jnp.bitwise_and(j, 3)
            si = idxb.at[2 * r4]
            di = idxb.at[2 * r4 + 1]
            pltpu.async_copy(s2_h.at[di], sdstb.at[slot], gsems[slot])
            pltpu.async_copy(whf_h.at[si], whb.at[slot], gsems[slot])

        def gather_wait(j, slot):
            r4 = jnp.bitwise_and(j, 3)
            si = idxb.at[2 * r4]
            di = idxb.at[2 * r4 + 1]
            pltpu.make_async_copy(s2_h.at[di], sdstb.at[slot],
                                  gsems[slot]).wait()
            pltpu.make_async_copy(whf_h.at[si], whb.at[slot],
                                  gsems[slot]).wait()

        def scatter_start(j, slot):
            r4 = jnp.bitwise_and(j, 3)
            di = idxb.at[2 * r4 + 1]
            h1 = pltpu.async_copy(whb.at[slot], nsh.at[di], ssems[slot],
                                  add=True)
            return (h1,)

        def compute(slot):
            db = sdstb.at[slot]
            whs = whb.at[slot]

            @pl.loop(0, B, step=4)
            def _edge(e0):
                for u in range(4):
                    e = e0 + u
                    t = whs[e, pl.ds(NF, 16)] + db[e, :]
                    t = jnp.where(t > 0, t, ALPHA * t)
                    w = jnp.exp(t)
                    whs[e, pl.ds(NF, 16)] = w
                    wv = _dyn_gather(w, lo_idx)
                    for h in range(H):
                        v = whs[e, pl.ds(h * OUT, OUT)]
                        whs[e, pl.ds(h * OUT, OUT)] = v * wv

        NPAIR = NCHUNK // 2
        stage_wait(0, 0)
        gather_start(0, 0)
        stage_wait(1, 1)
        gather_start(1, 1)

        @pl.loop(0, NPAIR)
        def _pair(p):
            j0 = 2 * p
            j1 = j0 + 1
            gather_wait(j0, 0)
            compute(0)
            sc0 = scatter_start(j0, 0)
            stage_wait(j0 + 2, 0)
            gather_start(j0 + 2, 0)
            gather_wait(j1, 1)
            compute(1)
            sc1 = scatter_start(j1, 1)

            @pl.when(p < NPAIR - 1)
            def _():
                stage_wait(j1 + 2, 1)
                gather_start(j1 + 2, 1)

            sc0[0].wait()
            sc1[0].wait()

            @pl.when(p < NPAIR - 1)
            def _():
                stage_start(j0 + 4, 0)

            @pl.when(p < NPAIR - 2)
            def _():
                stage_start(j1 + 4, 1)

        gather_wait(NCHUNK - 1, 0)
        compute(0)
        sce = scatter_start(NCHUNK - 1, 0)
        sce[0].wait()

        plsc.subcore_barrier()
        pltpu.sync_copy(nsh.at[pl.ds(rbase, RPT)],
                        nump_h.at[c].at[pl.ds(rbase, RPT)])

    return k(ei, s2_tab, whf, z144)


def _combine_body(np_ref, e_ref, p_ref, o_ref):
    m = np_ref[0] + np_ref[1]
    num = m[:, :NF]
    den = m[:, NF:]
    den = jnp.where(den == 0.0, 1.0, den)
    rec = 1.0 / den
    res = num * jnp.dot(rec, e_ref[...],
                        preferred_element_type=jnp.float32,
                        precision=lax.Precision.HIGHEST)
    o_ref[...] = jnp.dot(res, p_ref[...],
                         preferred_element_type=jnp.float32,
                         precision=lax.Precision.HIGHEST)


def _tc_combine(nump, eexp, pmat):
    mb = 1000
    return pl.pallas_call(
        _combine_body,
        grid=(N // mb,),
        in_specs=[
            pl.BlockSpec((NC, mb, NF + 16), lambda i: (0, i, 0)),
            pl.BlockSpec((16, NF), lambda i: (0, 0)),
            pl.BlockSpec((NF, NF), lambda i: (0, 0)),
        ],
        out_specs=pl.BlockSpec((mb, NF), lambda i: (i, 0)),
        out_shape=jax.ShapeDtypeStruct((N, NF), jnp.float32),
    )(nump, eexp, pmat)


def kernel(x, edge_index, W, W_b, a_w, a_b):
    wcat = jnp.transpose(W, (1, 0, 2)).reshape(NF, NF)
    bcat = W_b.reshape(1, NF)
    kcol = jnp.arange(NF, dtype=jnp.int32)
    perm = (kcol % H) * OUT + kcol // H
    wcat2 = wcat[:, perm]
    bcat2 = bcat[:, perm]
    asrc, adst = a_w[:, :OUT], a_w[:, OUT:]
    eye8 = jnp.eye(H, dtype=jnp.float32)
    a1h = (asrc.T[:, :, None] * eye8[None, :, :]).reshape(NF, H)
    a2h = (adst.T[:, :, None] * eye8[None, :, :]).reshape(NF, H)
    zc = jnp.zeros((NF, H), jnp.float32)
    a1 = jnp.concatenate([a1h, zc], axis=1)
    a2 = jnp.concatenate([a2h, zc], axis=1)
    b1 = jnp.concatenate([a_b, jnp.zeros((H,), jnp.float32)]).reshape(1, 16)
    b2 = jnp.concatenate([jnp.zeros((H,), jnp.float32),
                          jnp.full((H,), -1e30, jnp.float32)]).reshape(1, 16)
    eexp = jnp.concatenate(
        [jnp.tile(eye8, (1, OUT)),
         jnp.zeros((8, NF), jnp.float32)], axis=0)
    pmat = jax.nn.one_hot(perm, NF, dtype=jnp.float32)
    src = edge_index[0].astype(jnp.int32).reshape(NW, NCHUNK, B)
    dst = edge_index[1].astype(jnp.int32).reshape(NW, NCHUNK, B)
    ei = jnp.stack([src, dst], axis=2)
    z144 = jnp.zeros((N, NF + 16), jnp.float32)

    whf, s2_tab = _tc_project(x, wcat2, bcat2, a1, b1, a2, b2)
    nump = _sc_edge_pass(ei, s2_tab, whf, z144)
    return _tc_combine(nump, eexp, pmat)
